# baseline (speedup 1.0000x reference)
import functools

import jax
import jax.numpy as jnp
from jax import lax
from jax.experimental import pallas as pl
from jax.experimental.pallas import tpu as pltpu
from jax.experimental.pallas import tpu_sc as plsc

KV = 27
E_PER_K = 23000
C = 128
EPS = 1e-5
N_VOX = 50000

NC = 2
NS = 16

EPK_PAD = 23040
E_FLAT = KV * EPK_PAD
E_PAD = 627200
BIG = 1 << 30

RANGE = 12544
N_OUT = 4 * RANGE
DUMP = RANGE
ACC_ROWS = RANGE + 16
ROWS_PER_TILE = RANGE // NS

MM_BLK = 512
MM_J = EPK_PAD // MM_BLK

_mesh = plsc.VectorSubcoreMesh(
    core_axis_name="c", subcore_axis_name="s", num_cores=NC, num_subcores=NS)
_sc_params = pltpu.CompilerParams(needs_layout_passes=False)

GCH = 112
GSLOTS = 5
GBLK = GCH * GSLOTS


@functools.partial(
    pl.kernel,
    mesh=_mesh,
    out_type=jax.ShapeDtypeStruct((E_PAD, C), jnp.float32),
    scratch_types=[
        pltpu.VMEM((GBLK,), jnp.int32),
        pltpu.VMEM((GBLK, C), jnp.float32),
        pltpu.SemaphoreType.DMA,
    ] + [pltpu.SemaphoreType.DMA] * GSLOTS,
    compiler_params=_sc_params,
)
def _sc_gather(feat_hbm, src_hbm, out_hbm, idx_v, rows_v, gsem, *ssems):
    wid = lax.axis_index("s") * NC + lax.axis_index("c")
    n_blk = E_PAD // (NC * NS * GBLK)
    base = wid * n_blk * GBLK

    def body(i, carry):
        off = base + i * GBLK
        pltpu.sync_copy(src_hbm.at[pl.ds(off, GBLK)], idx_v)
        cps = []
        for b in range(GSLOTS):
            @pl.when(i > 0)
            def _(b=b):
                pltpu.make_async_copy(
                    out_hbm.at[pl.ds(0, GCH)],
                    rows_v.at[pl.ds(b * GCH, GCH)], ssems[b]).wait()
            cps.append(pltpu.async_copy(
                feat_hbm.at[idx_v.at[pl.ds(b * GCH, GCH)]],
                rows_v.at[pl.ds(b * GCH, GCH)], gsem))
        for b in range(GSLOTS):
            cps[b].wait()
            pltpu.async_copy(rows_v.at[pl.ds(b * GCH, GCH)],
                             out_hbm.at[pl.ds(off + b * GCH, GCH)], ssems[b])
        return carry

    lax.fori_loop(0, n_blk, body, 0)
    for b in range(GSLOTS):
        pltpu.make_async_copy(out_hbm.at[pl.ds(0, GCH)],
                              rows_v.at[pl.ds(b * GCH, GCH)], ssems[b]).wait()


def _mm_body(g_ref, w_ref, o_ref):
    o_ref[...] = jnp.dot(g_ref[...].astype(jnp.bfloat16), w_ref[0],
                         preferred_element_type=jnp.float32)


def _batched_mm(gathered, weight):
    return pl.pallas_call(
        _mm_body,
        grid=(KV, MM_J),
        in_specs=[
            pl.BlockSpec((MM_BLK, C), lambda k, j: (k * MM_J + j, 0)),
            pl.BlockSpec((1, C, C), lambda k, j: (k, 0, 0)),
        ],
        out_specs=pl.BlockSpec((MM_BLK, C), lambda k, j: (k * MM_J + j, 0)),
        out_shape=jax.ShapeDtypeStruct((E_PAD, C), jnp.float32),
    )(gathered, weight)


SCH = 48
IDXBLK = 1568
PER_TILE = E_PAD // NS
NBLK = PER_TILE // IDXBLK
CB_CAP = IDXBLK + 2 * SCH


@functools.partial(
    pl.kernel,
    mesh=_mesh,
    out_type=jax.ShapeDtypeStruct((N_OUT, C), jnp.float32),
    scratch_types=[
        pltpu.VMEM((IDXBLK,), jnp.int32),
        pltpu.VMEM((CB_CAP,), jnp.int32),
        pltpu.VMEM((CB_CAP,), jnp.int32),
        pltpu.VMEM((CB_CAP,), jnp.int32),
        pltpu.VMEM((CB_CAP,), jnp.int32),
        pltpu.VMEM((SCH,), jnp.int32),
        pltpu.VMEM((SCH, C), jnp.float32),
        pltpu.VMEM((SCH, C), jnp.float32),
        pltpu.VMEM_SHARED((ACC_ROWS, C), jnp.float32),
        pltpu.SemaphoreType.DMA,
        pltpu.SemaphoreType.DMA,
        pltpu.SemaphoreType.DMA,
        pltpu.SemaphoreType.DMA,
    ],
    compiler_params=_sc_params,
)
def _sc_scatter(msgs_hbm, dst_hbm, zeros_hbm, out_hbm,
                idx_v, eid0, eid1, ldst0, ldst1, dmp, rows0, rows1, acc,
                rsem0, rsem1, ssem0, ssem1):
    cid = lax.axis_index("c")
    sid = lax.axis_index("s")
    ebase = sid * PER_TILE
    lane = jnp.arange(16, dtype=jnp.int32)
    dump16 = jnp.full((16,), DUMP, jnp.int32)
    zero16 = jnp.zeros((16,), jnp.int32)
    slots = ((rows0, rsem0, ssem0), (rows1, rsem1, ssem1))
    cbufs = ((eid0, ldst0), (eid1, ldst1))

    for u in range(SCH // 16):
        dmp[pl.ds(16 * u, 16)] = dump16

    for p in range(2):
        rng = 2 * p + cid
        lo = rng * RANGE

        pltpu.sync_copy(zeros_hbm, acc.at[pl.ds(sid * ROWS_PER_TILE,
                                                ROWS_PER_TILE)])
        plsc.subcore_barrier()

        for rows, _, ssem in slots:
            pltpu.async_copy(rows, acc.at[dmp], ssem, add=True)

        def blk_body(i, eid_b, ldst_b, carry):
            off = ebase + i * IDXBLK
            pltpu.sync_copy(dst_hbm.at[pl.ds(off, IDXBLK)], idx_v)

            def vec(j, b):
                d = idx_v[pl.ds(j * 16, 16)]
                ok = (d >= lo) & (d < lo + RANGE)
                inc = plsc.cumsum(jnp.where(ok, 1, 0))
                pos = b + inc - 1
                plsc.store_scatter(ldst_b, [pos], d - lo, mask=ok)
                plsc.store_scatter(eid_b, [pos], off + j * 16 + lane,
                                   mask=ok)
                return b + inc[15]

            count = lax.fori_loop(0, IDXBLK // 16, vec, jnp.int32(0))

            for u in range(2 * SCH // 16):
                tail = count + u * 16 + lane
                plsc.store_scatter(eid_b, [tail], zero16)
                plsc.store_scatter(ldst_b, [tail], dump16)

            def pair(s, c):
                cps = []
                for par in range(2):
                    rows, rsem, ssem = slots[par]
                    coff = (2 * s + par) * SCH
                    pltpu.make_async_copy(msgs_hbm.at[pl.ds(0, SCH)], rows,
                                          ssem).wait()
                    cps.append(pltpu.async_copy(
                        msgs_hbm.at[eid_b.at[pl.ds(coff, SCH)]], rows, rsem))
                for par in range(2):
                    rows, rsem, ssem = slots[par]
                    coff = (2 * s + par) * SCH
                    cps[par].wait()
                    pltpu.async_copy(rows,
                                     acc.at[ldst_b.at[pl.ds(coff, SCH)]],
                                     ssem, add=True)
                return c

            n_pairs = jnp.maximum(1, (count + 2 * SCH - 1) // (2 * SCH))
            lax.fori_loop(0, n_pairs, pair, carry)
            return carry

        def body(i2, carry):
            i = 2 * i2
            blk_body(i, *cbufs[0], carry)
            blk_body(i + 1, *cbufs[1], carry)
            return carry

        lax.fori_loop(0, NBLK // 2, body, 0)
        blk_body(NBLK - 1, *cbufs[0], 0)

        for rows, _, ssem in slots:
            pltpu.make_async_copy(msgs_hbm.at[pl.ds(0, SCH)], rows,
                                  ssem).wait()
        plsc.subcore_barrier()

        pltpu.sync_copy(
            acc.at[pl.ds(sid * ROWS_PER_TILE, ROWS_PER_TILE)],
            out_hbm.at[pl.ds(lo + sid * ROWS_PER_TILE, ROWS_PER_TILE)])
        plsc.subcore_barrier()


def _stats_body(x_ref, s_ref, q_ref):
    pid = pl.program_id(0)
    row = lax.broadcasted_iota(jnp.int32, (MM_BLK, 1), 0) + pid * MM_BLK
    x = jnp.where(row < N_VOX, x_ref[...], 0.0)

    @pl.when(pid == 0)
    def _():
        s_ref[...] = jnp.zeros_like(s_ref)
        q_ref[...] = jnp.zeros_like(q_ref)

    s_ref[...] += jnp.sum(x, axis=0, keepdims=True)
    q_ref[...] += jnp.sum(x * x, axis=0, keepdims=True)


def _apply_body(x_ref, s_ref, q_ref, w_ref, b_ref, o_ref):
    inv_n = 1.0 / N_VOX
    mean = s_ref[...] * inv_n
    var = q_ref[...] * inv_n - mean * mean
    scale = lax.rsqrt(var + EPS) * w_ref[...]
    o_ref[...] = jnp.maximum((x_ref[...] - mean) * scale + b_ref[...], 0.0)


def _bn_relu(out_acc, bn_weight, bn_bias):
    nblk = N_OUT // MM_BLK
    s, q = pl.pallas_call(
        _stats_body,
        grid=(nblk,),
        in_specs=[pl.BlockSpec((MM_BLK, C), lambda i: (i, 0))],
        out_specs=[pl.BlockSpec((1, C), lambda i: (0, 0)),
                   pl.BlockSpec((1, C), lambda i: (0, 0))],
        out_shape=[jax.ShapeDtypeStruct((1, C), jnp.float32),
                   jax.ShapeDtypeStruct((1, C), jnp.float32)],
    )(out_acc)
    return pl.pallas_call(
        _apply_body,
        grid=(nblk,),
        in_specs=[
            pl.BlockSpec((MM_BLK, C), lambda i: (i, 0)),
            pl.BlockSpec((1, C), lambda i: (0, 0)),
            pl.BlockSpec((1, C), lambda i: (0, 0)),
            pl.BlockSpec((1, C), lambda i: (0, 0)),
            pl.BlockSpec((1, C), lambda i: (0, 0)),
        ],
        out_specs=pl.BlockSpec((MM_BLK, C), lambda i: (i, 0)),
        out_shape=jax.ShapeDtypeStruct((N_OUT, C), jnp.float32),
    )(out_acc, s, q, bn_weight.reshape(1, C), bn_bias.reshape(1, C))


def kernel(input_feat, input_coord, input_cmap, input_kmap, weight,
           bn_weight, bn_bias):
    src = input_kmap[0].reshape(KV, E_PER_K)
    dst = input_kmap[1].reshape(KV, E_PER_K)

    src_p = jnp.zeros((KV, EPK_PAD), jnp.int32).at[:, :E_PER_K].set(src)
    src_p = jnp.concatenate(
        [src_p.reshape(-1), jnp.zeros((E_PAD - E_FLAT,), jnp.int32)])
    dst_p = jnp.full((KV, EPK_PAD), BIG, jnp.int32).at[:, :E_PER_K].set(dst)
    dst_p = jnp.concatenate(
        [dst_p.reshape(-1), jnp.full((E_PAD - E_FLAT,), BIG, jnp.int32)])

    gathered = _sc_gather(input_feat, src_p)
    msgs = _batched_mm(gathered, weight.astype(jnp.bfloat16))
    zeros = jnp.zeros((ROWS_PER_TILE, C), jnp.float32)
    out_acc = _sc_scatter(msgs, dst_p, zeros)
    y = _bn_relu(out_acc, bn_weight, bn_bias)
    return y[:N_VOX]


# speedup vs baseline: 1.8168x; 1.8168x over previous
import functools

import jax
import jax.numpy as jnp
from jax import lax
from jax.experimental import pallas as pl
from jax.experimental.pallas import tpu as pltpu
from jax.experimental.pallas import tpu_sc as plsc

KV = 27
E_PER_K = 23000
C = 128
EPS = 1e-5
N_VOX = 50000

NC = 2
NS = 16

EPK_PAD = 23040
E_FLAT = KV * EPK_PAD
E_PAD = 627200
BIG = 1 << 30

RANGE = 12544
N_OUT = 4 * RANGE
DUMP = RANGE
ACC_ROWS = RANGE + 16
ROWS_PER_TILE = RANGE // NS

MM_BLK = 512
MM_J = EPK_PAD // MM_BLK

_mesh = plsc.VectorSubcoreMesh(
    core_axis_name="c", subcore_axis_name="s", num_cores=NC, num_subcores=NS)
_sc_params = pltpu.CompilerParams(needs_layout_passes=False)

GCH = 112
GSLOTS = 5
GBLK = GCH * GSLOTS
@functools.partial(
    pl.kernel,
    mesh=_mesh,
    out_type=jax.ShapeDtypeStruct((E_PAD, C), jnp.float32),
    scratch_types=[
        pltpu.VMEM((GBLK,), jnp.int32),
        pltpu.VMEM((GBLK, C), jnp.float32),
        pltpu.SemaphoreType.DMA,
    ] + [pltpu.SemaphoreType.DMA] * GSLOTS,
    compiler_params=_sc_params,
)
def _sc_gather(feat_hbm, src_hbm, out_hbm, idx_v, rows_v, gsem, *ssems):
    wid = lax.axis_index("s") * NC + lax.axis_index("c")
    n_blk = E_PAD // (NC * NS * GBLK)
    base = wid * n_blk * GBLK

    def body(i, carry):
        off = base + i * GBLK
        pltpu.sync_copy(src_hbm.at[pl.ds(off, GBLK)], idx_v)
        cps = []
        for b in range(GSLOTS):
            @pl.when(i > 0)
            def _(b=b):
                pltpu.make_async_copy(
                    out_hbm.at[pl.ds(0, GCH)],
                    rows_v.at[pl.ds(b * GCH, GCH)], ssems[b]).wait()
            cps.append(pltpu.async_copy(
                feat_hbm.at[idx_v.at[pl.ds(b * GCH, GCH)]],
                rows_v.at[pl.ds(b * GCH, GCH)], gsem))
        for b in range(GSLOTS):
            cps[b].wait()
            pltpu.async_copy(rows_v.at[pl.ds(b * GCH, GCH)],
                             out_hbm.at[pl.ds(off + b * GCH, GCH)], ssems[b])
        return carry

    lax.fori_loop(0, n_blk, body, 0)
    for b in range(GSLOTS):
        pltpu.make_async_copy(out_hbm.at[pl.ds(0, GCH)],
                              rows_v.at[pl.ds(b * GCH, GCH)], ssems[b]).wait()


def _mm_body(g_ref, w_ref, o_ref):
    o_ref[...] = jnp.dot(g_ref[...].astype(jnp.bfloat16), w_ref[0],
                         preferred_element_type=jnp.float32)


def _batched_mm(gathered, weight):
    return pl.pallas_call(
        _mm_body,
        grid=(KV, MM_J),
        in_specs=[
            pl.BlockSpec((MM_BLK, C), lambda k, j: (k * MM_J + j, 0)),
            pl.BlockSpec((1, C, C), lambda k, j: (k, 0, 0)),
        ],
        out_specs=pl.BlockSpec((MM_BLK, C), lambda k, j: (k * MM_J + j, 0)),
        out_shape=jax.ShapeDtypeStruct((E_PAD, C), jnp.float32),
    )(gathered, weight)


SCH = 112
IDXBLK = 1568


@functools.partial(
    pl.kernel,
    mesh=_mesh,
    out_type=jax.ShapeDtypeStruct((N_OUT, C), jnp.float32),
    scratch_types=[
        pltpu.VMEM((IDXBLK,), jnp.int32),
        pltpu.VMEM((SCH,), jnp.int32),
        pltpu.VMEM((SCH,), jnp.int32),
        pltpu.VMEM((SCH, C), jnp.float32),
        pltpu.VMEM((SCH, C), jnp.float32),
        pltpu.VMEM_SHARED((ACC_ROWS, C), jnp.float32),
        pltpu.SemaphoreType.DMA,
        pltpu.SemaphoreType.DMA,
        pltpu.SemaphoreType.DMA,
        pltpu.SemaphoreType.DMA,
    ],
    compiler_params=_sc_params,
)
def _sc_scatter(msgs_hbm, dst_hbm, zeros_hbm, out_hbm,
                idx_v, ld0, ld1, rows0, rows1, acc,
                rsem0, rsem1, ssem0, ssem1):
    cid = lax.axis_index("c")
    sid = lax.axis_index("s")
    per_tile = E_PAD // NS
    n_blk = per_tile // IDXBLK
    ebase = sid * per_tile
    dump16 = jnp.full((16,), DUMP, jnp.int32)
    slots = ((ld0, rows0, rsem0, ssem0), (ld1, rows1, rsem1, ssem1))

    for p in range(2):
        rng = 2 * p + cid
        lo = rng * RANGE

        pltpu.sync_copy(zeros_hbm, acc.at[pl.ds(sid * ROWS_PER_TILE,
                                                ROWS_PER_TILE)])
        plsc.subcore_barrier()

        for ld, rows, _, ssem in slots:
            for u in range(SCH // 16):
                ld[pl.ds(16 * u, 16)] = dump16
            pltpu.async_copy(rows, acc.at[ld], ssem, add=True)

        def chunk(s, par, off):
            ld, rows, rsem, ssem = slots[par]
            local = s * 2 * SCH + par * SCH
            pltpu.make_async_copy(msgs_hbm.at[pl.ds(0, SCH)], rows,
                                  ssem).wait()
            cp = pltpu.async_copy(msgs_hbm.at[pl.ds(off + local, SCH)],
                                  rows, rsem)
            for u in range(SCH // 16):
                d = idx_v[pl.ds(local + 16 * u, 16)]
                ok = (d >= lo) & (d < lo + RANGE)
                ld[pl.ds(16 * u, 16)] = jnp.where(ok, d - lo, DUMP)
            cp.wait()
            pltpu.async_copy(rows, acc.at[ld], ssem, add=True)

        def body(i, carry):
            off = ebase + i * IDXBLK
            pltpu.sync_copy(dst_hbm.at[pl.ds(off, IDXBLK)], idx_v)

            def pair(s, c):
                chunk(s, 0, off)
                chunk(s, 1, off)
                return c

            lax.fori_loop(0, IDXBLK // (2 * SCH), pair, carry)
            return carry

        lax.fori_loop(0, n_blk, body, 0)
        for ld, rows, _, ssem in slots:
            pltpu.make_async_copy(msgs_hbm.at[pl.ds(0, SCH)], rows,
                                  ssem).wait()
        plsc.subcore_barrier()

        pltpu.sync_copy(
            acc.at[pl.ds(sid * ROWS_PER_TILE, ROWS_PER_TILE)],
            out_hbm.at[pl.ds(lo + sid * ROWS_PER_TILE, ROWS_PER_TILE)])
        plsc.subcore_barrier()


def _stats_body(x_ref, s_ref, q_ref):
    pid = pl.program_id(0)
    row = lax.broadcasted_iota(jnp.int32, (MM_BLK, 1), 0) + pid * MM_BLK
    x = jnp.where(row < N_VOX, x_ref[...].astype(jnp.float32), 0.0)

    @pl.when(pid == 0)
    def _():
        s_ref[...] = jnp.zeros_like(s_ref)
        q_ref[...] = jnp.zeros_like(q_ref)

    s_ref[...] += jnp.sum(x, axis=0, keepdims=True)
    q_ref[...] += jnp.sum(x * x, axis=0, keepdims=True)


def _apply_body(x_ref, s_ref, q_ref, w_ref, b_ref, o_ref):
    inv_n = 1.0 / N_VOX
    mean = s_ref[...] * inv_n
    var = q_ref[...] * inv_n - mean * mean
    scale = lax.rsqrt(var + EPS) * w_ref[...]
    o_ref[...] = jnp.maximum(
        (x_ref[...].astype(jnp.float32) - mean) * scale + b_ref[...], 0.0)


def _bn_relu(out_acc, bn_weight, bn_bias):
    nblk = N_OUT // MM_BLK
    s, q = pl.pallas_call(
        _stats_body,
        grid=(nblk,),
        in_specs=[pl.BlockSpec((MM_BLK, C), lambda i: (i, 0))],
        out_specs=[pl.BlockSpec((1, C), lambda i: (0, 0)),
                   pl.BlockSpec((1, C), lambda i: (0, 0))],
        out_shape=[jax.ShapeDtypeStruct((1, C), jnp.float32),
                   jax.ShapeDtypeStruct((1, C), jnp.float32)],
    )(out_acc)
    return pl.pallas_call(
        _apply_body,
        grid=(nblk,),
        in_specs=[
            pl.BlockSpec((MM_BLK, C), lambda i: (i, 0)),
            pl.BlockSpec((1, C), lambda i: (0, 0)),
            pl.BlockSpec((1, C), lambda i: (0, 0)),
            pl.BlockSpec((1, C), lambda i: (0, 0)),
            pl.BlockSpec((1, C), lambda i: (0, 0)),
        ],
        out_specs=pl.BlockSpec((MM_BLK, C), lambda i: (i, 0)),
        out_shape=jax.ShapeDtypeStruct((N_OUT, C), jnp.float32),
    )(out_acc, s, q, bn_weight.reshape(1, C), bn_bias.reshape(1, C))


def kernel(input_feat, input_coord, input_cmap, input_kmap, weight,
           bn_weight, bn_bias):
    src = input_kmap[0].reshape(KV, E_PER_K)
    dst = input_kmap[1].reshape(KV, E_PER_K)

    src_p = jnp.zeros((KV, EPK_PAD), jnp.int32).at[:, :E_PER_K].set(src)
    src_p = jnp.concatenate(
        [src_p.reshape(-1), jnp.zeros((E_PAD - E_FLAT,), jnp.int32)])
    dst_p = jnp.full((KV, EPK_PAD), BIG, jnp.int32).at[:, :E_PER_K].set(dst)
    dst_p = jnp.concatenate(
        [dst_p.reshape(-1), jnp.full((E_PAD - E_FLAT,), BIG, jnp.int32)])

    gathered = _sc_gather(input_feat, src_p)
    msgs = _batched_mm(gathered, weight.astype(jnp.bfloat16))
    zeros = jnp.zeros((ROWS_PER_TILE, C), jnp.float32)
    out_acc = _sc_scatter(msgs, dst_p, zeros)
    y = _bn_relu(out_acc, bn_weight, bn_bias)
    return y[:N_VOX]
